# Initial kernel scaffold; baseline (speedup 1.0000x reference)
import jax
import jax.numpy as jnp
from jax.experimental import pallas as pl


def kernel(pred_bitrate, gcc_bitrate, fec_table, frame_samples, loss_flags, loss_counts, delay_gradient, fec_bins):
    raise NotImplementedError("write your pallas kernel here")


# speedup vs baseline: 3806.4434x; 3806.4434x over previous
import jax
import jax.numpy as jnp
from jax.experimental import pallas as pl


def _tc_kernel(pred_ref, gcc_ref, dg_ref, F_ref, bins_ref, fs_ref, lc_ref,
               mk_ref, out_ref):
    d = pred_ref[...] - gcc_ref[...]
    w = dg_ref[...]
    pos = jnp.maximum(d, 0.0)
    neg = jnp.maximum(-d, 0.0)
    br = jnp.sum(pos * pos * w + neg * neg * (1.0 - w),
                 keepdims=True).reshape(1, 1) * (1.0 / d.size)

    denom = jnp.maximum(jnp.sum(mk_ref[...], keepdims=True).reshape(1, 1), 1.0)

    F = F_ref[...]
    bins_col = bins_ref[...]
    iota32 = jax.lax.broadcasted_iota(jnp.int32, (32, 128), 0)
    rows = fs_ref.shape[0]

    def body(i, acc):
        fs_row = fs_ref[pl.ds(i, 1), :]
        lc_row = lc_ref[pl.ds(i, 1), :]
        mk_row = mk_ref[pl.ds(i, 1), :]
        alr = jnp.where(mk_row != 0.0, lc_row / fs_row, 0.0)
        cmp = (bins_col <= fs_row).astype(jnp.int32)
        bin_row = jnp.sum(cmp, axis=0, keepdims=True)
        oneh = (iota32 == bin_row).astype(jnp.float32)
        pf = jnp.dot(F, oneh, preferred_element_type=jnp.float32)
        dd = alr - pf
        colsum = jnp.sum(dd + 2.0 * jnp.abs(dd), axis=0, keepdims=True)
        return acc + mk_row * colsum

    acc = jax.lax.fori_loop(0, rows, body, jnp.zeros((1, 128), jnp.float32))
    s = jnp.sum(acc, keepdims=True)
    out_ref[...] = br + s / denom


def kernel(pred_bitrate, gcc_bitrate, fec_table, frame_samples, loss_flags,
           loss_counts, delay_gradient, fec_bins):
    B, NBINS = fec_table.shape
    J = frame_samples.size
    ROWS = J // 128
    fs = frame_samples.reshape(ROWS, 128).astype(jnp.float32)
    lc = loss_counts.reshape(ROWS, 128).astype(jnp.float32)
    mk = (loss_flags.reshape(ROWS, 128) != 0).astype(jnp.float32)
    pred2 = pred_bitrate.reshape(8, 128)
    gcc2 = gcc_bitrate.reshape(8, 128)
    dg2 = delay_gradient.reshape(8, 128)
    bins_pad = jnp.concatenate(
        [fec_bins.astype(jnp.float32),
         jnp.full((NBINS - fec_bins.shape[0],), jnp.inf, jnp.float32)]
    ).reshape(NBINS, 1)

    out = pl.pallas_call(
        _tc_kernel,
        out_shape=jax.ShapeDtypeStruct((1, 1), jnp.float32),
    )(pred2, gcc2, dg2, fec_table, bins_pad, fs, lc, mk)
    return out[0, 0]


# speedup vs baseline: 9882.4141x; 2.5962x over previous
import jax
import jax.numpy as jnp
from jax.experimental import pallas as pl

_JB = 2048


def _tc_kernel(pred_ref, gcc_ref, dg_ref, F_ref, bins_ref, fs_ref, lc_ref,
               mk_ref, out_ref):
    d = pred_ref[...] - gcc_ref[...]
    w = dg_ref[...]
    pos = jnp.maximum(d, 0.0)
    neg = jnp.maximum(-d, 0.0)
    br = jnp.sum(pos * pos * w + neg * neg * (1.0 - w),
                 keepdims=True).reshape(1, 1) * (1.0 / d.size)

    denom = jnp.maximum(jnp.sum(mk_ref[...], keepdims=True).reshape(1, 1), 1.0)

    F = F_ref[...]
    B = F.shape[0]
    colsum = jnp.sum(F, axis=0, keepdims=True)
    ones_row = jnp.ones((1, B), jnp.float32)
    bins_col = bins_ref[...]
    iota32 = jax.lax.broadcasted_iota(jnp.int32, (32, _JB), 0)
    steps = fs_ref.shape[0]

    acc = jnp.zeros((1, _JB), jnp.float32)
    for i in range(steps):
        fs_row = fs_ref[i:i + 1, :]
        lc_row = lc_ref[i:i + 1, :]
        mk_row = mk_ref[i:i + 1, :]
        alr = jnp.where(mk_row != 0.0, lc_row / fs_row, 0.0)
        cmp = (bins_col <= fs_row).astype(jnp.int32)
        bin_row = jnp.sum(cmp, axis=0, keepdims=True)
        oneh = (iota32 == bin_row).astype(jnp.float32)
        pf = jnp.dot(F, oneh, preferred_element_type=jnp.float32)
        absd = jnp.abs(alr - pf)
        colabs = jnp.dot(ones_row, absd,
                         preferred_element_type=jnp.float32)
        lin = jnp.dot(colsum, oneh,
                      preferred_element_type=jnp.float32)
        acc = acc + mk_row * (2.0 * colabs + (float(B) * alr - lin))

    s = jnp.sum(acc, keepdims=True)
    out_ref[...] = br + s / denom


def kernel(pred_bitrate, gcc_bitrate, fec_table, frame_samples, loss_flags,
           loss_counts, delay_gradient, fec_bins):
    B, NBINS = fec_table.shape
    J = frame_samples.size
    STEPS = J // _JB
    fs = frame_samples.reshape(STEPS, _JB).astype(jnp.float32)
    lc = loss_counts.reshape(STEPS, _JB).astype(jnp.float32)
    mk = (loss_flags.reshape(STEPS, _JB) != 0).astype(jnp.float32)
    pred2 = pred_bitrate.reshape(8, 128)
    gcc2 = gcc_bitrate.reshape(8, 128)
    dg2 = delay_gradient.reshape(8, 128)
    bins_pad = jnp.concatenate(
        [fec_bins.astype(jnp.float32),
         jnp.full((NBINS - fec_bins.shape[0],), jnp.inf, jnp.float32)]
    ).reshape(NBINS, 1)

    out = pl.pallas_call(
        _tc_kernel,
        out_shape=jax.ShapeDtypeStruct((1, 1), jnp.float32),
    )(pred2, gcc2, dg2, fec_table, bins_pad, fs, lc, mk)
    return out[0, 0]
